# Initial kernel scaffold; baseline (speedup 1.0000x reference)
import jax
import jax.numpy as jnp
from jax.experimental import pallas as pl


def kernel(conv64, conv128, conv256, verts, params, edges):
    raise NotImplementedError("write your pallas kernel here")


# speedup vs baseline: 7.2390x; 7.2390x over previous
import functools

import jax
import jax.numpy as jnp
from jax import lax
from jax.experimental import pallas as pl
from jax.experimental.pallas import tpu as pltpu
from jax.experimental.pallas import tpu_sc as plsc

N_VERTS = 50000
N_EDGES = 800000
HID = 64
OUT_PAD = 16

NC, NS = 2, 16
NW = NC * NS
NP = 50176
NV_W = NP // NW
NR_T = NP // NS
VQ = NP // 4
NR_Q = VQ // NS

CHUNK = 128
K = 8
GRP = K * CHUNK
EPAD = 1605632
EROWS = EPAD // CHUNK
N_ITEMS = 1638400
IROWS = N_ITEMS // CHUNK
N_PADI = N_ITEMS - 2 * N_EDGES
EPAD_EXT = EPAD + N_PADI
EROWS_EXT = EPAD_EXT // CHUNK

PKBITS = 14

_mesh = plsc.VectorSubcoreMesh(core_axis_name="c", subcore_axis_name="s")
_SC_PARAMS = pltpu.CompilerParams(use_tc_tiling_on_sc=False)


def _mm2(x_parts, W0, b0, W1, b1, bilin=False):
    if bilin:
        wx, wy, r00, r01, r10, r11 = x_parts
        ins = [wx.reshape(NP, 1), wy.reshape(NP, 1), r00, r01, r10, r11]
        d = int(r00.shape[1])
        nx = 6
    else:
        ins = list(x_parts)
        d = sum(int(x.shape[1]) for x in x_parts)
        nx = len(x_parts)
    w = int(W0.shape[1])
    BR = 1024
    grid = (NP // BR,)

    def body(*refs):
        xr = refs[:nx]
        w0r, b0r, w1r, b1r = refs[nx:nx + 4]
        outs = refs[nx + 4:]
        if bilin:
            wxv = xr[0][...]
            wyv = xr[1][...]
            f00 = xr[2][...]
            f01 = xr[3][...]
            f10 = xr[4][...]
            f11 = xr[5][...]
            top = f00 + wxv * (f01 - f00)
            bot = f10 + wxv * (f11 - f10)
            xs = [top + wyv * (bot - top)]
        else:
            xs = [r[...] for r in xr]
        h0 = b0r[...].astype(jnp.float32)
        h1 = b1r[...].astype(jnp.float32)
        r0 = 0
        for xp in xs:
            dp = xp.shape[1]
            h0 = h0 + jnp.dot(xp, w0r[r0:r0 + dp, :],
                              preferred_element_type=jnp.float32)
            h1 = h1 + jnp.dot(xp, w1r[r0:r0 + dp, :],
                              preferred_element_type=jnp.float32)
            r0 += dp
        row = (pl.program_id(0) * BR
               + lax.broadcasted_iota(jnp.int32, (BR, 1), 0))
        h1 = jnp.where(row < N_VERTS, h1, 0.0)
        outs[0][...] = h0
        outs[1][...] = h1

    in_specs = [pl.BlockSpec((BR, int(x.shape[1])), lambda i: (i, 0))
                for x in ins]
    in_specs += [
        pl.BlockSpec((d, w), lambda i: (0, 0)),
        pl.BlockSpec((1, w), lambda i: (0, 0)),
        pl.BlockSpec((d, w), lambda i: (0, 0)),
        pl.BlockSpec((1, w), lambda i: (0, 0)),
    ]
    out_shape = [jax.ShapeDtypeStruct((NP, w), jnp.float32)] * 2
    out_specs = [pl.BlockSpec((BR, w), lambda i: (i, 0))] * 2

    return pl.pallas_call(
        body, grid=grid, in_specs=in_specs, out_specs=out_specs,
        out_shape=out_shape,
    )(*ins, W0, b0.reshape(1, w), W1, b1.reshape(1, w))


@functools.partial(
    pl.kernel,
    out_type=jax.ShapeDtypeStruct((EPAD_EXT,), jnp.int32),
    mesh=_mesh,
    compiler_params=_SC_PARAMS,
    scratch_types=[
        pltpu.VMEM((K, CHUNK), jnp.int32),
        pltpu.VMEM((K, CHUNK), jnp.int32),
        pltpu.SemaphoreType.DMA,
        pltpu.SemaphoreType.DMA,
    ],
)
def _part_scatter(pos, vals, out, pbuf, vbuf, semi, sems):
    c = lax.axis_index("c")
    s = lax.axis_index("s")
    wid = s * NC + c
    base = wid * (IROWS // NW)

    @pl.loop(0, IROWS // NW, step=K)
    def _(off):
        b = base + off
        d1 = pltpu.async_copy(pos.at[pl.ds(b, K)], pbuf, semi)
        pltpu.async_copy(vals.at[pl.ds(b, K)], vbuf, semi).wait()
        d1.wait()
        sd = [pltpu.async_copy(vbuf.at[j], out.at[pbuf.at[j]], sems)
              for j in range(K)]
        for dsc in sd:
            dsc.wait()


@functools.partial(
    pl.kernel,
    out_type=jax.ShapeDtypeStruct((NP, HID), jnp.float32),
    mesh=_mesh,
    compiler_params=_SC_PARAMS,
    scratch_types=[
        pltpu.VMEM((16,), jnp.int32),
        pltpu.VMEM((K, CHUNK), jnp.int32),
        pltpu.VMEM((K, CHUNK), jnp.int32),
        pltpu.VMEM((K, CHUNK), jnp.int32),
        pltpu.VMEM((K, CHUNK, HID), jnp.float32),
        pltpu.VMEM_SHARED((VQ, HID), jnp.float32),
        pltpu.SemaphoreType.DMA,
        pltpu.SemaphoreType.DMA,
        pltpu.SemaphoreType.DMA,
    ],
)
def _agg(h0, h1, pk, bounds, out, bnd, pkb, sidx, didx, rows, acc,
         semg, sems, semi):
    c = lax.axis_index("c")
    s = lax.axis_index("s")

    pltpu.sync_copy(bounds, bnd)
    bv = bnd[pl.ds(0, 16)]

    for pp in range(2):
        srow = jnp.where(c == 0, bv[pp], bv[2 + pp])
        ng = jnp.where(c == 0, bv[4 + pp], bv[6 + pp])
        qq = 2 * c + pp
        v0 = qq * VQ + s * NR_Q
        accsl = pl.ds(s * NR_Q, NR_Q)
        pltpu.sync_copy(h0.at[pl.ds(v0, NR_Q)], acc.at[accsl])
        plsc.subcore_barrier()

        @pl.loop(s, ng, step=NS)
        def _(g):
            b = srow + g * K
            pltpu.async_copy(pk.at[pl.ds(b, K)], pkb, semi).wait()
            for j in range(K):
                for t in range(CHUNK // 16):
                    sl = pl.ds(t * 16, 16)
                    w = pkb[j, sl]
                    didx[j, sl] = w & ((1 << PKBITS) - 1)
                    sidx[j, sl] = lax.shift_right_logical(w, PKBITS)
            gd = [pltpu.async_copy(h1.at[sidx.at[j]], rows.at[j], semg)
                  for j in range(K)]
            sd = []
            for j in range(K):
                gd[j].wait()
                sd.append(pltpu.async_copy(rows.at[j], acc.at[didx.at[j]],
                                           sems, add=True))
            for dsc in sd:
                dsc.wait()

        plsc.subcore_barrier()
        pltpu.sync_copy(acc.at[accsl], out.at[pl.ds(v0, NR_Q)])
        plsc.subcore_barrier()


@functools.partial(
    pl.kernel,
    out_type=jax.ShapeDtypeStruct((2, NP, OUT_PAD), jnp.float32),
    mesh=_mesh,
    compiler_params=_SC_PARAMS,
    scratch_types=[
        pltpu.VMEM((K, CHUNK), jnp.int32),
        pltpu.VMEM((K, CHUNK), jnp.int32),
        pltpu.VMEM((K, CHUNK, OUT_PAD), jnp.float32),
        pltpu.VMEM_SHARED((NP, OUT_PAD), jnp.float32),
        pltpu.SemaphoreType.DMA,
        pltpu.SemaphoreType.DMA,
        pltpu.SemaphoreType.DMA,
    ],
)
def _agg_out(h0p, zerosp, h1p, srcs, dsts, out,
             sidx, didx, rows, acc, semg, sems, semi):
    c = lax.axis_index("c")
    s = lax.axis_index("s")
    rowsl = pl.ds(s * NR_T, NR_T)

    @pl.when(c == 0)
    def _():
        pltpu.sync_copy(h0p.at[rowsl], acc.at[rowsl])

    @pl.when(c == 1)
    def _():
        pltpu.sync_copy(zerosp.at[rowsl], acc.at[rowsl])

    plsc.subcore_barrier()

    grows = EROWS // NW
    base = (c * NS + s) * grows

    @pl.loop(0, grows, step=K)
    def _(off):
        b = base + off
        di = pltpu.async_copy(srcs.at[pl.ds(b, K)], sidx, semi)
        pltpu.async_copy(dsts.at[pl.ds(b, K)], didx, semi).wait()
        di.wait()
        gd = [pltpu.async_copy(h1p.at[sidx.at[j]], rows.at[j], semg)
              for j in range(K)]
        sd = []
        for j in range(K):
            gd[j].wait()
            sd.append(pltpu.async_copy(rows.at[j], acc.at[didx.at[j]],
                                       sems, add=True))
        for dsc in sd:
            dsc.wait()

    plsc.subcore_barrier()
    pltpu.sync_copy(acc.at[rowsl], out.at[c, rowsl])


VC = 112


def _valign_body(H, W, C, refs):
    featT, px, py, wxo, wyo, o00, o01, o10, o11 = refs[:9]
    (pxb, pyb, i00, i01, i10, i11, wxb, wyb,
     r00, r01, r10, r11, semg) = refs[9:]

    c = lax.axis_index("c")
    s = lax.axis_index("s")
    wid = s * NC + c
    v0 = wid * NV_W

    fW = float(W)
    fH = float(H)

    @pl.loop(0, NV_W, step=VC)
    def _(voff):
        vb = v0 + voff
        pltpu.sync_copy(px.at[pl.ds(vb, VC)], pxb)
        pltpu.sync_copy(py.at[pl.ds(vb, VC)], pyb)

        for j in range(VC // 16):
            sl = pl.ds(j * 16, 16)
            gx = (pxb[sl] + 1.0) * (0.5 * (fW - 1.0))
            gy = (pyb[sl] + 1.0) * (0.5 * (fH - 1.0))
            gx = jnp.minimum(jnp.maximum(gx, 0.0), fW - 1.0)
            gy = jnp.minimum(jnp.maximum(gy, 0.0), fH - 1.0)
            x0 = jnp.minimum(gx.astype(jnp.int32), W - 2)
            y0 = jnp.minimum(gy.astype(jnp.int32), H - 2)
            wxb[sl] = gx - x0.astype(jnp.float32)
            wyb[sl] = gy - y0.astype(jnp.float32)
            base = y0 * W + x0
            i00[sl] = base
            i01[sl] = base + 1
            i10[sl] = base + W
            i11[sl] = base + W + 1

        g0 = pltpu.async_copy(featT.at[i00], r00, semg)
        g1 = pltpu.async_copy(featT.at[i01], r01, semg)
        g2 = pltpu.async_copy(featT.at[i10], r10, semg)
        g3 = pltpu.async_copy(featT.at[i11], r11, semg)
        pltpu.sync_copy(wxb, wxo.at[pl.ds(vb, VC)])
        pltpu.sync_copy(wyb, wyo.at[pl.ds(vb, VC)])
        vsl = pl.ds(vb, VC)
        g0.wait()
        pltpu.sync_copy(r00, o00.at[vsl])
        g1.wait()
        pltpu.sync_copy(r01, o01.at[vsl])
        g2.wait()
        pltpu.sync_copy(r10, o10.at[vsl])
        g3.wait()
        pltpu.sync_copy(r11, o11.at[vsl])


def _make_valign(H, W, C):
    out_type = (
        [jax.ShapeDtypeStruct((NP,), jnp.float32)] * 2
        + [jax.ShapeDtypeStruct((NP, C), jnp.float32)] * 4
    )
    scratch = [
        pltpu.VMEM((VC,), jnp.float32),
        pltpu.VMEM((VC,), jnp.float32),
        pltpu.VMEM((VC,), jnp.int32),
        pltpu.VMEM((VC,), jnp.int32),
        pltpu.VMEM((VC,), jnp.int32),
        pltpu.VMEM((VC,), jnp.int32),
        pltpu.VMEM((VC,), jnp.float32),
        pltpu.VMEM((VC,), jnp.float32),
        pltpu.VMEM((VC, C), jnp.float32),
        pltpu.VMEM((VC, C), jnp.float32),
        pltpu.VMEM((VC, C), jnp.float32),
        pltpu.VMEM((VC, C), jnp.float32),
        pltpu.SemaphoreType.DMA,
    ]

    def body(*refs):
        _valign_body(H, W, C, refs)

    return pl.kernel(body, out_type=out_type, mesh=_mesh,
                     scratch_types=scratch,
                     compiler_params=_SC_PARAMS)


_valign_s1 = _make_valign(64, 64, 64)
_valign_s2 = _make_valign(32, 32, 128)
_valign_s3 = _make_valign(16, 16, 128)


def _conv(x_parts, W0, b0, W1, b1, pk, bounds, bilin=False):
    h0, h1 = _mm2(x_parts, W0, b0, W1, b1, bilin=bilin)
    return _agg(h0, h1, pk, bounds)


def _conv_out(x_parts, W0, b0, W1, b1, zerosp, srcs, dsts):
    W0p = jnp.pad(W0, ((0, 0), (0, OUT_PAD - W0.shape[1])))
    b0p = jnp.pad(b0, (0, OUT_PAD - b0.shape[0]))
    W1p = jnp.pad(W1, ((0, 0), (0, OUT_PAD - W1.shape[1])))
    b1p = jnp.pad(b1, (0, OUT_PAD - b1.shape[0]))
    h0p, h1p = _mm2(x_parts, W0p, b0p, W1p, b1p)
    return _agg_out(h0p, zerosp, h1p, srcs, dsts)


def _stage(xb, prefix, p, zerosp, eidx):
    pk, bounds, srcs_u, dsts_u = eidx
    x = _conv(xb, p[prefix + '_in_W0'], p[prefix + '_in_b0'],
              p[prefix + '_in_W1'], p[prefix + '_in_b1'],
              pk, bounds, bilin=True)
    for i in range(12):
        x = _conv([x], p[prefix + '_hid_W0'][i], p[prefix + '_hid_b0'][i],
                  p[prefix + '_hid_W1'][i], p[prefix + '_hid_b1'][i],
                  pk, bounds)
    return _conv_out([x], p[prefix + '_out_W0'], p[prefix + '_out_b0'],
                     p[prefix + '_out_W1'], p[prefix + '_out_b1'],
                     zerosp, srcs_u, dsts_u)


def kernel(conv64, conv128, conv256, verts, params, edges):
    e0 = edges[:, 0].astype(jnp.int32)
    e1 = edges[:, 1].astype(jnp.int32)
    npad = EPAD - 2 * N_EDGES
    fill = jnp.full((npad,), NP - 1, jnp.int32)
    srcs_u = jnp.concatenate([e1, e0, fill]).reshape(EROWS, CHUNK)
    dsts_u = jnp.concatenate([e0, e1, fill]).reshape(EROWS, CHUNK)

    dirs_s = jnp.concatenate([e1, e0])
    dirs_d = jnp.concatenate([e0, e1])
    q = dirs_d // VQ
    NB = 512
    MB = (2 * N_EDGES) // NB
    ranks = []
    nums = []
    for k in range(4):
        qk = (q == k).astype(jnp.int32).reshape(MB, NB)
        wb = jnp.cumsum(qk, axis=1)
        bs = wb[:, -1]
        off = jnp.cumsum(bs) - bs
        ranks.append((off[:, None] + wb - 1).reshape(-1))
        nums.append(jnp.sum(bs))
    n = jnp.stack(nums)
    caps = ((n + (GRP - 1)) // GRP) * GRP
    base = jnp.cumsum(caps) - caps
    rank = (jnp.where(q == 0, ranks[0], 0) + jnp.where(q == 1, ranks[1], 0)
            + jnp.where(q == 2, ranks[2], 0) + jnp.where(q == 3, ranks[3], 0))
    pos_real = base[q] + rank
    ii = jnp.arange(N_PADI, dtype=jnp.int32)
    g0 = caps[0] - n[0]
    g1 = caps[1] - n[1]
    g2 = caps[2] - n[2]
    g3 = caps[3] - n[3]
    e_used = base[3] + caps[3]
    g4 = EPAD - e_used
    pos_pad = jnp.where(
        ii < g0, base[0] + n[0] + ii,
        jnp.where(ii < g0 + g1, base[1] + n[1] + (ii - g0),
        jnp.where(ii < g0 + g1 + g2, base[2] + n[2] + (ii - g0 - g1),
        jnp.where(ii < g0 + g1 + g2 + g3,
                  base[3] + n[3] + (ii - g0 - g1 - g2),
        jnp.where(ii < g0 + g1 + g2 + g3 + g4,
                  e_used + (ii - g0 - g1 - g2 - g3),
                  EPAD + (ii - g0 - g1 - g2 - g3 - g4))))))
    dstl = dirs_d - q * VQ
    pk_real = (dirs_s << PKBITS) | dstl
    pk_fill = jnp.full((N_PADI,), (NP - 1) << PKBITS, jnp.int32)
    pos_all = jnp.concatenate([pos_real, pos_pad]).reshape(IROWS, CHUNK)
    pk_all = jnp.concatenate([pk_real, pk_fill]).reshape(IROWS, CHUNK)
    pk = _part_scatter(pos_all, pk_all).reshape(EROWS_EXT, CHUNK)
    bounds = jnp.concatenate(
        [base // CHUNK, (n + (GRP - 1)) // GRP,
         jnp.zeros((8,), jnp.int32)]).astype(jnp.int32)
    eidx = (pk, bounds, srcs_u, dsts_u)

    featT1 = conv64[0].reshape(64, 64 * 64).T
    featT2 = conv128[0].reshape(128, 32 * 32).T
    featT3 = conv256[0].reshape(128, 16 * 16).T

    posv = jnp.pad(verts, ((0, NP - N_VERTS), (0, 0)), constant_values=-1.0)
    zerosp = jnp.zeros((NP, OUT_PAD), jnp.float32)

    xb1 = _valign_s1(featT1, posv[:, 0], posv[:, 1])
    v1 = _stage(xb1, 's1', params, zerosp, eidx)
    p1 = v1[0] + v1[1]
    xb2 = _valign_s2(featT2, p1[:, 0], p1[:, 1])
    v2 = _stage(xb2, 's2', params, zerosp, eidx)
    p2 = v2[0] + v2[1]
    xb3 = _valign_s3(featT3, p2[:, 0], p2[:, 1])
    v3 = _stage(xb3, 's3', params, zerosp, eidx)

    return (v3[0] + v3[1])[:N_VERTS, :3]


# speedup vs baseline: 9.0111x; 1.2448x over previous
import functools

import jax
import jax.numpy as jnp
from jax import lax
from jax.experimental import pallas as pl
from jax.experimental.pallas import tpu as pltpu
from jax.experimental.pallas import tpu_sc as plsc

N_VERTS = 50000
N_EDGES = 800000
HID = 64
OUT_PAD = 16

NC, NS = 2, 16
NW = NC * NS
NP = 50176
NV_W = NP // NW
NR_T = NP // NS
VQ = NP // 4
NR_Q = VQ // NS

CHUNK = 128
K = 8
GRP = K * CHUNK
EPAD = 1605632
EROWS = EPAD // CHUNK
N_ITEMS = 1638400
IROWS = N_ITEMS // CHUNK
N_PADI = N_ITEMS - 2 * N_EDGES
EPAD_EXT = EPAD + N_PADI
EROWS_EXT = EPAD_EXT // CHUNK

PKBITS = 14

_mesh = plsc.VectorSubcoreMesh(core_axis_name="c", subcore_axis_name="s")
_SC_PARAMS = pltpu.CompilerParams(use_tc_tiling_on_sc=False)


def _mm2(x_parts, W0, b0, W1, b1, bilin=False):
    if bilin:
        wx, wy, r00, r01, r10, r11 = x_parts
        ins = [wx.reshape(NP, 1), wy.reshape(NP, 1), r00, r01, r10, r11]
        d = int(r00.shape[1])
        nx = 6
    else:
        ins = list(x_parts)
        d = sum(int(x.shape[1]) for x in x_parts)
        nx = len(x_parts)
    w = int(W0.shape[1])
    BR = 3584
    grid = (NP // BR,)

    def body(*refs):
        xr = refs[:nx]
        w0r, b0r, w1r, b1r = refs[nx:nx + 4]
        outs = refs[nx + 4:]
        if bilin:
            wxv = xr[0][...]
            wyv = xr[1][...]
            f00 = xr[2][...]
            f01 = xr[3][...]
            f10 = xr[4][...]
            f11 = xr[5][...]
            top = f00 + wxv * (f01 - f00)
            bot = f10 + wxv * (f11 - f10)
            xs = [top + wyv * (bot - top)]
        else:
            xs = [r[...] for r in xr]
        h0 = b0r[...].astype(jnp.float32)
        h1 = b1r[...].astype(jnp.float32)
        r0 = 0
        for xp in xs:
            dp = xp.shape[1]
            h0 = h0 + jnp.dot(xp, w0r[r0:r0 + dp, :],
                              preferred_element_type=jnp.float32)
            h1 = h1 + jnp.dot(xp, w1r[r0:r0 + dp, :],
                              preferred_element_type=jnp.float32)
            r0 += dp
        row = (pl.program_id(0) * BR
               + lax.broadcasted_iota(jnp.int32, (BR, 1), 0))
        h1 = jnp.where(row < N_VERTS, h1, 0.0)
        outs[0][...] = h0
        outs[1][...] = h1

    in_specs = [pl.BlockSpec((BR, int(x.shape[1])), lambda i: (i, 0))
                for x in ins]
    in_specs += [
        pl.BlockSpec((d, w), lambda i: (0, 0)),
        pl.BlockSpec((1, w), lambda i: (0, 0)),
        pl.BlockSpec((d, w), lambda i: (0, 0)),
        pl.BlockSpec((1, w), lambda i: (0, 0)),
    ]
    out_shape = [jax.ShapeDtypeStruct((NP, w), jnp.float32)] * 2
    out_specs = [pl.BlockSpec((BR, w), lambda i: (i, 0))] * 2

    return pl.pallas_call(
        body, grid=grid, in_specs=in_specs, out_specs=out_specs,
        out_shape=out_shape,
    )(*ins, W0, b0.reshape(1, w), W1, b1.reshape(1, w))


@functools.partial(
    pl.kernel,
    out_type=jax.ShapeDtypeStruct((EPAD_EXT,), jnp.int32),
    mesh=_mesh,
    compiler_params=_SC_PARAMS,
    scratch_types=[
        pltpu.VMEM((K, CHUNK), jnp.int32),
        pltpu.VMEM((K, CHUNK), jnp.int32),
        pltpu.SemaphoreType.DMA,
        pltpu.SemaphoreType.DMA,
    ],
)
def _part_scatter(pos, vals, out, pbuf, vbuf, semi, sems):
    c = lax.axis_index("c")
    s = lax.axis_index("s")
    wid = s * NC + c
    base = wid * (IROWS // NW)

    @pl.loop(0, IROWS // NW, step=K)
    def _(off):
        b = base + off
        d1 = pltpu.async_copy(pos.at[pl.ds(b, K)], pbuf, semi)
        pltpu.async_copy(vals.at[pl.ds(b, K)], vbuf, semi).wait()
        d1.wait()
        sd = [pltpu.async_copy(vbuf.at[j], out.at[pbuf.at[j]], sems)
              for j in range(K)]
        for dsc in sd:
            dsc.wait()


@functools.partial(
    pl.kernel,
    out_type=jax.ShapeDtypeStruct((NP, HID), jnp.float32),
    mesh=_mesh,
    compiler_params=_SC_PARAMS,
    scratch_types=[
        pltpu.VMEM((16,), jnp.int32),
        pltpu.VMEM((K, CHUNK), jnp.int32),
        pltpu.VMEM((K, CHUNK), jnp.int32),
        pltpu.VMEM((K, CHUNK), jnp.int32),
        pltpu.VMEM((K, CHUNK, HID), jnp.float32),
        pltpu.VMEM_SHARED((VQ, HID), jnp.float32),
        pltpu.SemaphoreType.DMA,
        pltpu.SemaphoreType.DMA,
        pltpu.SemaphoreType.DMA,
    ],
)
def _agg(h0, h1, pk, bounds, out, bnd, pkb, sidx, didx, rows, acc,
         semg, sems, semi):
    c = lax.axis_index("c")
    s = lax.axis_index("s")

    pltpu.sync_copy(bounds, bnd)
    bv = bnd[pl.ds(0, 16)]

    for pp in range(2):
        srow = jnp.where(c == 0, bv[pp], bv[2 + pp])
        ng = jnp.where(c == 0, bv[4 + pp], bv[6 + pp])
        qq = 2 * c + pp
        v0 = qq * VQ + s * NR_Q
        accsl = pl.ds(s * NR_Q, NR_Q)
        pltpu.sync_copy(h0.at[pl.ds(v0, NR_Q)], acc.at[accsl])
        plsc.subcore_barrier()

        @pl.loop(s, ng, step=NS)
        def _(g):
            b = srow + g * K
            pltpu.async_copy(pk.at[pl.ds(b, K)], pkb, semi).wait()
            for j in range(K):
                for t in range(CHUNK // 16):
                    sl = pl.ds(t * 16, 16)
                    w = pkb[j, sl]
                    didx[j, sl] = w & ((1 << PKBITS) - 1)
                    sidx[j, sl] = lax.shift_right_logical(w, PKBITS)
            gd = [pltpu.async_copy(h1.at[sidx.at[j]], rows.at[j], semg)
                  for j in range(K)]
            sd = []
            for j in range(K):
                gd[j].wait()
                sd.append(pltpu.async_copy(rows.at[j], acc.at[didx.at[j]],
                                           sems, add=True))
            for dsc in sd:
                dsc.wait()

        plsc.subcore_barrier()
        pltpu.sync_copy(acc.at[accsl], out.at[pl.ds(v0, NR_Q)])
        plsc.subcore_barrier()


@functools.partial(
    pl.kernel,
    out_type=jax.ShapeDtypeStruct((2, NP, OUT_PAD), jnp.float32),
    mesh=_mesh,
    compiler_params=_SC_PARAMS,
    scratch_types=[
        pltpu.VMEM((K, CHUNK), jnp.int32),
        pltpu.VMEM((K, CHUNK), jnp.int32),
        pltpu.VMEM((K, CHUNK, OUT_PAD), jnp.float32),
        pltpu.VMEM_SHARED((NP, OUT_PAD), jnp.float32),
        pltpu.SemaphoreType.DMA,
        pltpu.SemaphoreType.DMA,
        pltpu.SemaphoreType.DMA,
    ],
)
def _agg_out(h0p, zerosp, h1p, srcs, dsts, out,
             sidx, didx, rows, acc, semg, sems, semi):
    c = lax.axis_index("c")
    s = lax.axis_index("s")
    rowsl = pl.ds(s * NR_T, NR_T)

    @pl.when(c == 0)
    def _():
        pltpu.sync_copy(h0p.at[rowsl], acc.at[rowsl])

    @pl.when(c == 1)
    def _():
        pltpu.sync_copy(zerosp.at[rowsl], acc.at[rowsl])

    plsc.subcore_barrier()

    grows = EROWS // NW
    base = (c * NS + s) * grows

    @pl.loop(0, grows, step=K)
    def _(off):
        b = base + off
        di = pltpu.async_copy(srcs.at[pl.ds(b, K)], sidx, semi)
        pltpu.async_copy(dsts.at[pl.ds(b, K)], didx, semi).wait()
        di.wait()
        gd = [pltpu.async_copy(h1p.at[sidx.at[j]], rows.at[j], semg)
              for j in range(K)]
        sd = []
        for j in range(K):
            gd[j].wait()
            sd.append(pltpu.async_copy(rows.at[j], acc.at[didx.at[j]],
                                       sems, add=True))
        for dsc in sd:
            dsc.wait()

    plsc.subcore_barrier()
    pltpu.sync_copy(acc.at[rowsl], out.at[c, rowsl])


VC = 112


def _valign_body(H, W, C, refs):
    featT, px, py, wxo, wyo, o00, o01, o10, o11 = refs[:9]
    (pxb, pyb, i00, i01, i10, i11, wxb, wyb, shtab,
     ra, semt, semg, semw) = refs[9:]

    c = lax.axis_index("c")
    s = lax.axis_index("s")
    wid = s * NC + c
    v0 = wid * NV_W

    HW = H * W
    trows = HW // NS
    tsl = pl.ds(s * trows, trows)
    pltpu.async_copy(featT.at[tsl], shtab.at[tsl], semt).wait()

    dpx = pltpu.async_copy(px.at[pl.ds(v0, NV_W)], pxb, semt)
    pltpu.async_copy(py.at[pl.ds(v0, NV_W)], pyb, semt).wait()
    dpx.wait()

    fW = float(W)
    fH = float(H)

    @pl.loop(0, NV_W, step=16)
    def _(j0):
        sl = pl.ds(j0, 16)
        gx = (pxb[sl] + 1.0) * (0.5 * (fW - 1.0))
        gy = (pyb[sl] + 1.0) * (0.5 * (fH - 1.0))
        gx = jnp.minimum(jnp.maximum(gx, 0.0), fW - 1.0)
        gy = jnp.minimum(jnp.maximum(gy, 0.0), fH - 1.0)
        x0 = jnp.minimum(gx.astype(jnp.int32), W - 2)
        y0 = jnp.minimum(gy.astype(jnp.int32), H - 2)
        wxb[sl] = gx - x0.astype(jnp.float32)
        wyb[sl] = gy - y0.astype(jnp.float32)
        base = y0 * W + x0
        i00[sl] = base
        i01[sl] = base + 1
        i10[sl] = base + W
        i11[sl] = base + W + 1

    dwx = pltpu.async_copy(wxb, wxo.at[pl.ds(v0, NV_W)], semw)
    dwy = pltpu.async_copy(wyb, wyo.at[pl.ds(v0, NV_W)], semw)

    plsc.subcore_barrier()

    nchunks = NV_W // VC
    bufs = ra
    wdescs = [None] * nchunks
    for ci in range(nchunks):
        if ci >= 1:
            for dsc in wdescs[ci - 1]:
                dsc.wait()
        isl = pl.ds(ci * VC, VC)
        gd = [pltpu.async_copy(shtab.at[i00.at[isl]], bufs[0], semg),
              pltpu.async_copy(shtab.at[i01.at[isl]], bufs[1], semg),
              pltpu.async_copy(shtab.at[i10.at[isl]], bufs[2], semg),
              pltpu.async_copy(shtab.at[i11.at[isl]], bufs[3], semg)]
        for dsc in gd:
            dsc.wait()
        vsl = pl.ds(v0 + ci * VC, VC)
        wdescs[ci] = [pltpu.async_copy(bufs[0], o00.at[vsl], semw),
                      pltpu.async_copy(bufs[1], o01.at[vsl], semw),
                      pltpu.async_copy(bufs[2], o10.at[vsl], semw),
                      pltpu.async_copy(bufs[3], o11.at[vsl], semw)]
    for dsc in wdescs[nchunks - 1]:
        dsc.wait()
    dwx.wait()
    dwy.wait()


def _make_valign(H, W, C):
    out_type = (
        [jax.ShapeDtypeStruct((NP,), jnp.float32)] * 2
        + [jax.ShapeDtypeStruct((NP, C), jnp.float32)] * 4
    )
    rset = lambda: [pltpu.VMEM((VC, C), jnp.float32) for _ in range(4)]
    scratch = [
        pltpu.VMEM((NV_W,), jnp.float32),
        pltpu.VMEM((NV_W,), jnp.float32),
        pltpu.VMEM((NV_W,), jnp.int32),
        pltpu.VMEM((NV_W,), jnp.int32),
        pltpu.VMEM((NV_W,), jnp.int32),
        pltpu.VMEM((NV_W,), jnp.int32),
        pltpu.VMEM((NV_W,), jnp.float32),
        pltpu.VMEM((NV_W,), jnp.float32),
        pltpu.VMEM_SHARED((H * W, C), jnp.float32),
        rset(),
        pltpu.SemaphoreType.DMA,
        pltpu.SemaphoreType.DMA,
        pltpu.SemaphoreType.DMA,
    ]

    def body(*refs):
        _valign_body(H, W, C, refs)

    return pl.kernel(body, out_type=out_type, mesh=_mesh,
                     scratch_types=scratch,
                     compiler_params=_SC_PARAMS)


_valign_s1 = _make_valign(64, 64, 64)
_valign_s2 = _make_valign(32, 32, 128)
_valign_s3 = _make_valign(16, 16, 128)


def _conv(x_parts, W0, b0, W1, b1, pk, bounds, bilin=False):
    h0, h1 = _mm2(x_parts, W0, b0, W1, b1, bilin=bilin)
    return _agg(h0, h1, pk, bounds)


def _conv_out(x_parts, W0, b0, W1, b1, zerosp, srcs, dsts):
    W0p = jnp.pad(W0, ((0, 0), (0, OUT_PAD - W0.shape[1])))
    b0p = jnp.pad(b0, (0, OUT_PAD - b0.shape[0]))
    W1p = jnp.pad(W1, ((0, 0), (0, OUT_PAD - W1.shape[1])))
    b1p = jnp.pad(b1, (0, OUT_PAD - b1.shape[0]))
    h0p, h1p = _mm2(x_parts, W0p, b0p, W1p, b1p)
    return _agg_out(h0p, zerosp, h1p, srcs, dsts)


def _stage(xb, prefix, p, zerosp, eidx):
    pk, bounds, srcs_u, dsts_u = eidx
    x = _conv(xb, p[prefix + '_in_W0'], p[prefix + '_in_b0'],
              p[prefix + '_in_W1'], p[prefix + '_in_b1'],
              pk, bounds, bilin=True)
    for i in range(12):
        x = _conv([x], p[prefix + '_hid_W0'][i], p[prefix + '_hid_b0'][i],
                  p[prefix + '_hid_W1'][i], p[prefix + '_hid_b1'][i],
                  pk, bounds)
    return _conv_out([x], p[prefix + '_out_W0'], p[prefix + '_out_b0'],
                     p[prefix + '_out_W1'], p[prefix + '_out_b1'],
                     zerosp, srcs_u, dsts_u)


def kernel(conv64, conv128, conv256, verts, params, edges):
    e0 = edges[:, 0].astype(jnp.int32)
    e1 = edges[:, 1].astype(jnp.int32)
    npad = EPAD - 2 * N_EDGES
    fill = jnp.full((npad,), NP - 1, jnp.int32)
    srcs_u = jnp.concatenate([e1, e0, fill]).reshape(EROWS, CHUNK)
    dsts_u = jnp.concatenate([e0, e1, fill]).reshape(EROWS, CHUNK)

    dirs_s = jnp.concatenate([e1, e0])
    dirs_d = jnp.concatenate([e0, e1])
    q = dirs_d // VQ
    NB = 512
    MB = (2 * N_EDGES) // NB
    ranks = []
    nums = []
    for k in range(4):
        qk = (q == k).astype(jnp.int32).reshape(MB, NB)
        wb = jnp.cumsum(qk, axis=1)
        bs = wb[:, -1]
        off = jnp.cumsum(bs) - bs
        ranks.append((off[:, None] + wb - 1).reshape(-1))
        nums.append(jnp.sum(bs))
    n = jnp.stack(nums)
    caps = ((n + (GRP - 1)) // GRP) * GRP
    base = jnp.cumsum(caps) - caps
    rank = (jnp.where(q == 0, ranks[0], 0) + jnp.where(q == 1, ranks[1], 0)
            + jnp.where(q == 2, ranks[2], 0) + jnp.where(q == 3, ranks[3], 0))
    pos_real = base[q] + rank
    ii = jnp.arange(N_PADI, dtype=jnp.int32)
    g0 = caps[0] - n[0]
    g1 = caps[1] - n[1]
    g2 = caps[2] - n[2]
    g3 = caps[3] - n[3]
    e_used = base[3] + caps[3]
    g4 = EPAD - e_used
    pos_pad = jnp.where(
        ii < g0, base[0] + n[0] + ii,
        jnp.where(ii < g0 + g1, base[1] + n[1] + (ii - g0),
        jnp.where(ii < g0 + g1 + g2, base[2] + n[2] + (ii - g0 - g1),
        jnp.where(ii < g0 + g1 + g2 + g3,
                  base[3] + n[3] + (ii - g0 - g1 - g2),
        jnp.where(ii < g0 + g1 + g2 + g3 + g4,
                  e_used + (ii - g0 - g1 - g2 - g3),
                  EPAD + (ii - g0 - g1 - g2 - g3 - g4))))))
    dstl = dirs_d - q * VQ
    pk_real = (dirs_s << PKBITS) | dstl
    pk_fill = jnp.full((N_PADI,), (NP - 1) << PKBITS, jnp.int32)
    pos_all = jnp.concatenate([pos_real, pos_pad]).reshape(IROWS, CHUNK)
    pk_all = jnp.concatenate([pk_real, pk_fill]).reshape(IROWS, CHUNK)
    pk = _part_scatter(pos_all, pk_all).reshape(EROWS_EXT, CHUNK)
    bounds = jnp.concatenate(
        [base // CHUNK, (n + (GRP - 1)) // GRP,
         jnp.zeros((8,), jnp.int32)]).astype(jnp.int32)
    eidx = (pk, bounds, srcs_u, dsts_u)

    featT1 = conv64[0].reshape(64, 64 * 64).T
    featT2 = conv128[0].reshape(128, 32 * 32).T
    featT3 = conv256[0].reshape(128, 16 * 16).T

    posv = jnp.pad(verts, ((0, NP - N_VERTS), (0, 0)), constant_values=-1.0)
    zerosp = jnp.zeros((NP, OUT_PAD), jnp.float32)

    xb1 = _valign_s1(featT1, posv[:, 0], posv[:, 1])
    v1 = _stage(xb1, 's1', params, zerosp, eidx)
    p1 = v1[0] + v1[1]
    xb2 = _valign_s2(featT2, p1[:, 0], p1[:, 1])
    v2 = _stage(xb2, 's2', params, zerosp, eidx)
    p2 = v2[0] + v2[1]
    xb3 = _valign_s3(featT3, p2[:, 0], p2[:, 1])
    v3 = _stage(xb3, 's3', params, zerosp, eidx)

    return (v3[0] + v3[1])[:N_VERTS, :3]


# speedup vs baseline: 9.4380x; 1.0474x over previous
import functools

import jax
import jax.numpy as jnp
from jax import lax
from jax.experimental import pallas as pl
from jax.experimental.pallas import tpu as pltpu
from jax.experimental.pallas import tpu_sc as plsc

N_VERTS = 50000
N_EDGES = 800000
HID = 64
OUT_PAD = 16

NC, NS = 2, 16
NW = NC * NS
NP = 50176
NV_W = NP // NW
NR_T = NP // NS
VQ = NP // 4
NR_Q = VQ // NS

CHUNK = 128
K = 8
GRP = K * CHUNK
EPAD = 1605632
EROWS = EPAD // CHUNK
N_ITEMS = 1638400
IROWS = N_ITEMS // CHUNK
N_PADI = N_ITEMS - 2 * N_EDGES
EPAD_EXT = EPAD + N_PADI
EROWS_EXT = EPAD_EXT // CHUNK

PKBITS = 14

_mesh = plsc.VectorSubcoreMesh(core_axis_name="c", subcore_axis_name="s")
_SC_PARAMS = pltpu.CompilerParams(use_tc_tiling_on_sc=False)


def _mm2(x_parts, W0, b0, W1, b1, bilin=False):
    if bilin:
        wx, wy, r00, r01, r10, r11 = x_parts
        ins = [wx.reshape(NP, 1), wy.reshape(NP, 1), r00, r01, r10, r11]
        d = int(r00.shape[1])
        nx = 6
    else:
        ins = list(x_parts)
        d = sum(int(x.shape[1]) for x in x_parts)
        nx = len(x_parts)
    w = int(W0.shape[1])
    BR = 3584
    grid = (NP // BR,)

    def body(*refs):
        xr = refs[:nx]
        w0r, b0r, w1r, b1r = refs[nx:nx + 4]
        outs = refs[nx + 4:]
        if bilin:
            wxv = xr[0][...]
            wyv = xr[1][...]
            f00 = xr[2][...]
            f01 = xr[3][...]
            f10 = xr[4][...]
            f11 = xr[5][...]
            top = f00 + wxv * (f01 - f00)
            bot = f10 + wxv * (f11 - f10)
            xs = [top + wyv * (bot - top)]
        else:
            xs = [r[...] for r in xr]
        h0 = b0r[...].astype(jnp.float32)
        h1 = b1r[...].astype(jnp.float32)
        r0 = 0
        for xp in xs:
            dp = xp.shape[1]
            h0 = h0 + jnp.dot(xp, w0r[r0:r0 + dp, :],
                              preferred_element_type=jnp.float32)
            h1 = h1 + jnp.dot(xp, w1r[r0:r0 + dp, :],
                              preferred_element_type=jnp.float32)
            r0 += dp
        row = (pl.program_id(0) * BR
               + lax.broadcasted_iota(jnp.int32, (BR, 1), 0))
        h1 = jnp.where(row < N_VERTS, h1, 0.0)
        outs[0][...] = h0
        outs[1][...] = h1

    in_specs = [pl.BlockSpec((BR, int(x.shape[1])), lambda i: (i, 0))
                for x in ins]
    in_specs += [
        pl.BlockSpec((d, w), lambda i: (0, 0)),
        pl.BlockSpec((1, w), lambda i: (0, 0)),
        pl.BlockSpec((d, w), lambda i: (0, 0)),
        pl.BlockSpec((1, w), lambda i: (0, 0)),
    ]
    out_shape = [jax.ShapeDtypeStruct((NP, w), jnp.float32)] * 2
    out_specs = [pl.BlockSpec((BR, w), lambda i: (i, 0))] * 2

    return pl.pallas_call(
        body, grid=grid, in_specs=in_specs, out_specs=out_specs,
        out_shape=out_shape,
    )(*ins, W0, b0.reshape(1, w), W1, b1.reshape(1, w))


@functools.partial(
    pl.kernel,
    out_type=jax.ShapeDtypeStruct((EPAD_EXT,), jnp.int32),
    mesh=_mesh,
    compiler_params=_SC_PARAMS,
    scratch_types=[
        pltpu.VMEM((4, K, CHUNK), jnp.int32),
        pltpu.VMEM((4, K, CHUNK), jnp.int32),
        pltpu.SemaphoreType.DMA,
        pltpu.SemaphoreType.DMA,
    ],
)
def _part_scatter(pos, vals, out, pbuf, vbuf, semi, sems):
    c = lax.axis_index("c")
    s = lax.axis_index("s")
    wid = s * NC + c
    base = wid * (IROWS // NW)
    niter = IROWS // NW // K
    RB = 4

    def load(it):
        b = base + it * K
        return [pltpu.async_copy(pos.at[pl.ds(b, K)], pbuf.at[it % RB], semi),
                pltpu.async_copy(vals.at[pl.ds(b, K)], vbuf.at[it % RB], semi)]

    loads = [None] * niter
    scats = [None] * niter
    loads[0] = load(0)
    for it in range(niter):
        if it + 1 < niter:
            if it >= RB - 1:
                for dsc in scats[it - (RB - 1)]:
                    dsc.wait()
            loads[it + 1] = load(it + 1)
        for dsc in loads[it]:
            dsc.wait()
        scats[it] = [pltpu.async_copy(vbuf.at[it % RB, j],
                                      out.at[pbuf.at[it % RB, j]], sems)
                     for j in range(K)]
    for it in range(max(niter - RB, 0), niter):
        for dsc in scats[it]:
            dsc.wait()


@functools.partial(
    pl.kernel,
    out_type=jax.ShapeDtypeStruct((NP, HID), jnp.float32),
    mesh=_mesh,
    compiler_params=_SC_PARAMS,
    scratch_types=[
        pltpu.VMEM((16,), jnp.int32),
        pltpu.VMEM((K, CHUNK), jnp.int32),
        pltpu.VMEM((K, CHUNK), jnp.int32),
        pltpu.VMEM((K, CHUNK), jnp.int32),
        pltpu.VMEM((K, CHUNK, HID), jnp.float32),
        pltpu.VMEM_SHARED((VQ, HID), jnp.float32),
        pltpu.SemaphoreType.DMA,
        pltpu.SemaphoreType.DMA,
        pltpu.SemaphoreType.DMA,
    ],
)
def _agg(h0, h1, pk, bounds, out, bnd, pkb, sidx, didx, rows, acc,
         semg, sems, semi):
    c = lax.axis_index("c")
    s = lax.axis_index("s")

    pltpu.sync_copy(bounds, bnd)
    bv = bnd[pl.ds(0, 16)]

    for pp in range(2):
        srow = jnp.where(c == 0, bv[pp], bv[2 + pp])
        ng = jnp.where(c == 0, bv[4 + pp], bv[6 + pp])
        qq = 2 * c + pp
        v0 = qq * VQ + s * NR_Q
        accsl = pl.ds(s * NR_Q, NR_Q)
        pltpu.sync_copy(h0.at[pl.ds(v0, NR_Q)], acc.at[accsl])
        plsc.subcore_barrier()

        @pl.loop(s, ng, step=NS)
        def _(g):
            b = srow + g * K
            di = pltpu.async_copy(pk.at[pl.ds(b, K)], pkb, semi)

            @pl.when(g >= s + NS)
            def _():
                for j in range(K):
                    pltpu.make_async_copy(h1.at[pl.ds(0, CHUNK)],
                                          rows.at[j], sems).wait()

            di.wait()
            for j in range(K):
                for t in range(CHUNK // 16):
                    sl = pl.ds(t * 16, 16)
                    w = pkb[j, sl]
                    didx[j, sl] = w & ((1 << PKBITS) - 1)
                    sidx[j, sl] = lax.shift_right_logical(w, PKBITS)
            gd = [pltpu.async_copy(h1.at[sidx.at[j]], rows.at[j], semg)
                  for j in range(K)]
            for j in range(K):
                gd[j].wait()
                pltpu.async_copy(rows.at[j], acc.at[didx.at[j]],
                                 sems, add=True)

        @pl.when(ng > s)
        def _():
            for j in range(K):
                pltpu.make_async_copy(h1.at[pl.ds(0, CHUNK)],
                                      rows.at[j], sems).wait()

        plsc.subcore_barrier()
        pltpu.sync_copy(acc.at[accsl], out.at[pl.ds(v0, NR_Q)])
        plsc.subcore_barrier()


@functools.partial(
    pl.kernel,
    out_type=jax.ShapeDtypeStruct((2, NP, OUT_PAD), jnp.float32),
    mesh=_mesh,
    compiler_params=_SC_PARAMS,
    scratch_types=[
        pltpu.VMEM((K, CHUNK), jnp.int32),
        pltpu.VMEM((K, CHUNK), jnp.int32),
        pltpu.VMEM((K, CHUNK, OUT_PAD), jnp.float32),
        pltpu.VMEM_SHARED((NP, OUT_PAD), jnp.float32),
        pltpu.SemaphoreType.DMA,
        pltpu.SemaphoreType.DMA,
        pltpu.SemaphoreType.DMA,
    ],
)
def _agg_out(h0p, zerosp, h1p, srcs, dsts, out,
             sidx, didx, rows, acc, semg, sems, semi):
    c = lax.axis_index("c")
    s = lax.axis_index("s")
    rowsl = pl.ds(s * NR_T, NR_T)

    @pl.when(c == 0)
    def _():
        pltpu.sync_copy(h0p.at[rowsl], acc.at[rowsl])

    @pl.when(c == 1)
    def _():
        pltpu.sync_copy(zerosp.at[rowsl], acc.at[rowsl])

    plsc.subcore_barrier()

    grows = EROWS // NW
    base = (c * NS + s) * grows

    @pl.loop(0, grows, step=K)
    def _(off):
        b = base + off
        di = pltpu.async_copy(srcs.at[pl.ds(b, K)], sidx, semi)
        pltpu.async_copy(dsts.at[pl.ds(b, K)], didx, semi).wait()
        di.wait()
        gd = [pltpu.async_copy(h1p.at[sidx.at[j]], rows.at[j], semg)
              for j in range(K)]
        sd = []
        for j in range(K):
            gd[j].wait()
            sd.append(pltpu.async_copy(rows.at[j], acc.at[didx.at[j]],
                                       sems, add=True))
        for dsc in sd:
            dsc.wait()

    plsc.subcore_barrier()
    pltpu.sync_copy(acc.at[rowsl], out.at[c, rowsl])


VC = 112


def _valign_body(H, W, C, refs):
    featT, px, py, wxo, wyo, o00, o01, o10, o11 = refs[:9]
    (pxb, pyb, i00, i01, i10, i11, wxb, wyb, shtab,
     ra, semt, semg, semw) = refs[9:]

    c = lax.axis_index("c")
    s = lax.axis_index("s")
    wid = s * NC + c
    v0 = wid * NV_W

    HW = H * W
    trows = HW // NS
    tsl = pl.ds(s * trows, trows)
    pltpu.async_copy(featT.at[tsl], shtab.at[tsl], semt).wait()

    dpx = pltpu.async_copy(px.at[pl.ds(v0, NV_W)], pxb, semt)
    pltpu.async_copy(py.at[pl.ds(v0, NV_W)], pyb, semt).wait()
    dpx.wait()

    fW = float(W)
    fH = float(H)

    @pl.loop(0, NV_W, step=16)
    def _(j0):
        sl = pl.ds(j0, 16)
        gx = (pxb[sl] + 1.0) * (0.5 * (fW - 1.0))
        gy = (pyb[sl] + 1.0) * (0.5 * (fH - 1.0))
        gx = jnp.minimum(jnp.maximum(gx, 0.0), fW - 1.0)
        gy = jnp.minimum(jnp.maximum(gy, 0.0), fH - 1.0)
        x0 = jnp.minimum(gx.astype(jnp.int32), W - 2)
        y0 = jnp.minimum(gy.astype(jnp.int32), H - 2)
        wxb[sl] = gx - x0.astype(jnp.float32)
        wyb[sl] = gy - y0.astype(jnp.float32)
        base = y0 * W + x0
        i00[sl] = base
        i01[sl] = base + 1
        i10[sl] = base + W
        i11[sl] = base + W + 1

    dwx = pltpu.async_copy(wxb, wxo.at[pl.ds(v0, NV_W)], semw)
    dwy = pltpu.async_copy(wyb, wyo.at[pl.ds(v0, NV_W)], semw)

    plsc.subcore_barrier()

    nchunks = NV_W // VC
    bufs = ra
    wdescs = [None] * nchunks
    for ci in range(nchunks):
        if ci >= 1:
            for dsc in wdescs[ci - 1]:
                dsc.wait()
        isl = pl.ds(ci * VC, VC)
        gd = [pltpu.async_copy(shtab.at[i00.at[isl]], bufs[0], semg),
              pltpu.async_copy(shtab.at[i01.at[isl]], bufs[1], semg),
              pltpu.async_copy(shtab.at[i10.at[isl]], bufs[2], semg),
              pltpu.async_copy(shtab.at[i11.at[isl]], bufs[3], semg)]
        for dsc in gd:
            dsc.wait()
        vsl = pl.ds(v0 + ci * VC, VC)
        wdescs[ci] = [pltpu.async_copy(bufs[0], o00.at[vsl], semw),
                      pltpu.async_copy(bufs[1], o01.at[vsl], semw),
                      pltpu.async_copy(bufs[2], o10.at[vsl], semw),
                      pltpu.async_copy(bufs[3], o11.at[vsl], semw)]
    for dsc in wdescs[nchunks - 1]:
        dsc.wait()
    dwx.wait()
    dwy.wait()


def _make_valign(H, W, C):
    out_type = (
        [jax.ShapeDtypeStruct((NP,), jnp.float32)] * 2
        + [jax.ShapeDtypeStruct((NP, C), jnp.float32)] * 4
    )
    rset = lambda: [pltpu.VMEM((VC, C), jnp.float32) for _ in range(4)]
    scratch = [
        pltpu.VMEM((NV_W,), jnp.float32),
        pltpu.VMEM((NV_W,), jnp.float32),
        pltpu.VMEM((NV_W,), jnp.int32),
        pltpu.VMEM((NV_W,), jnp.int32),
        pltpu.VMEM((NV_W,), jnp.int32),
        pltpu.VMEM((NV_W,), jnp.int32),
        pltpu.VMEM((NV_W,), jnp.float32),
        pltpu.VMEM((NV_W,), jnp.float32),
        pltpu.VMEM_SHARED((H * W, C), jnp.float32),
        rset(),
        pltpu.SemaphoreType.DMA,
        pltpu.SemaphoreType.DMA,
        pltpu.SemaphoreType.DMA,
    ]

    def body(*refs):
        _valign_body(H, W, C, refs)

    return pl.kernel(body, out_type=out_type, mesh=_mesh,
                     scratch_types=scratch,
                     compiler_params=_SC_PARAMS)


_valign_s1 = _make_valign(64, 64, 64)
_valign_s2 = _make_valign(32, 32, 128)
_valign_s3 = _make_valign(16, 16, 128)


def _conv(x_parts, W0, b0, W1, b1, pk, bounds, bilin=False):
    h0, h1 = _mm2(x_parts, W0, b0, W1, b1, bilin=bilin)
    return _agg(h0, h1, pk, bounds)


def _conv_out(x_parts, W0, b0, W1, b1, zerosp, srcs, dsts):
    W0p = jnp.pad(W0, ((0, 0), (0, OUT_PAD - W0.shape[1])))
    b0p = jnp.pad(b0, (0, OUT_PAD - b0.shape[0]))
    W1p = jnp.pad(W1, ((0, 0), (0, OUT_PAD - W1.shape[1])))
    b1p = jnp.pad(b1, (0, OUT_PAD - b1.shape[0]))
    h0p, h1p = _mm2(x_parts, W0p, b0p, W1p, b1p)
    return _agg_out(h0p, zerosp, h1p, srcs, dsts)


def _stage(xb, prefix, p, zerosp, eidx):
    pk, bounds, srcs_u, dsts_u = eidx
    x = _conv(xb, p[prefix + '_in_W0'], p[prefix + '_in_b0'],
              p[prefix + '_in_W1'], p[prefix + '_in_b1'],
              pk, bounds, bilin=True)
    for i in range(12):
        x = _conv([x], p[prefix + '_hid_W0'][i], p[prefix + '_hid_b0'][i],
                  p[prefix + '_hid_W1'][i], p[prefix + '_hid_b1'][i],
                  pk, bounds)
    return _conv_out([x], p[prefix + '_out_W0'], p[prefix + '_out_b0'],
                     p[prefix + '_out_W1'], p[prefix + '_out_b1'],
                     zerosp, srcs_u, dsts_u)


def kernel(conv64, conv128, conv256, verts, params, edges):
    e0 = edges[:, 0].astype(jnp.int32)
    e1 = edges[:, 1].astype(jnp.int32)
    npad = EPAD - 2 * N_EDGES
    fill = jnp.full((npad,), NP - 1, jnp.int32)
    srcs_u = jnp.concatenate([e1, e0, fill]).reshape(EROWS, CHUNK)
    dsts_u = jnp.concatenate([e0, e1, fill]).reshape(EROWS, CHUNK)

    dirs_s = jnp.concatenate([e1, e0])
    dirs_d = jnp.concatenate([e0, e1])
    q = dirs_d // VQ
    NB = 512
    MB = (2 * N_EDGES) // NB
    ranks = []
    nums = []
    for k in range(4):
        qk = (q == k).astype(jnp.int32).reshape(MB, NB)
        wb = jnp.cumsum(qk, axis=1)
        bs = wb[:, -1]
        off = jnp.cumsum(bs) - bs
        ranks.append((off[:, None] + wb - 1).reshape(-1))
        nums.append(jnp.sum(bs))
    n = jnp.stack(nums)
    caps = ((n + (GRP - 1)) // GRP) * GRP
    base = jnp.cumsum(caps) - caps
    rank = (jnp.where(q == 0, ranks[0], 0) + jnp.where(q == 1, ranks[1], 0)
            + jnp.where(q == 2, ranks[2], 0) + jnp.where(q == 3, ranks[3], 0))
    pos_real = base[q] + rank
    ii = jnp.arange(N_PADI, dtype=jnp.int32)
    g0 = caps[0] - n[0]
    g1 = caps[1] - n[1]
    g2 = caps[2] - n[2]
    g3 = caps[3] - n[3]
    e_used = base[3] + caps[3]
    g4 = EPAD - e_used
    pos_pad = jnp.where(
        ii < g0, base[0] + n[0] + ii,
        jnp.where(ii < g0 + g1, base[1] + n[1] + (ii - g0),
        jnp.where(ii < g0 + g1 + g2, base[2] + n[2] + (ii - g0 - g1),
        jnp.where(ii < g0 + g1 + g2 + g3,
                  base[3] + n[3] + (ii - g0 - g1 - g2),
        jnp.where(ii < g0 + g1 + g2 + g3 + g4,
                  e_used + (ii - g0 - g1 - g2 - g3),
                  EPAD + (ii - g0 - g1 - g2 - g3 - g4))))))
    dstl = dirs_d - q * VQ
    pk_real = (dirs_s << PKBITS) | dstl
    pk_fill = jnp.full((N_PADI,), (NP - 1) << PKBITS, jnp.int32)
    pos_all = jnp.concatenate([pos_real, pos_pad]).reshape(IROWS, CHUNK)
    pk_all = jnp.concatenate([pk_real, pk_fill]).reshape(IROWS, CHUNK)
    pk = _part_scatter(pos_all, pk_all).reshape(EROWS_EXT, CHUNK)
    bounds = jnp.concatenate(
        [base // CHUNK, (n + (GRP - 1)) // GRP,
         jnp.zeros((8,), jnp.int32)]).astype(jnp.int32)
    eidx = (pk, bounds, srcs_u, dsts_u)

    featT1 = conv64[0].reshape(64, 64 * 64).T
    featT2 = conv128[0].reshape(128, 32 * 32).T
    featT3 = conv256[0].reshape(128, 16 * 16).T

    posv = jnp.pad(verts, ((0, NP - N_VERTS), (0, 0)), constant_values=-1.0)
    zerosp = jnp.zeros((NP, OUT_PAD), jnp.float32)

    xb1 = _valign_s1(featT1, posv[:, 0], posv[:, 1])
    v1 = _stage(xb1, 's1', params, zerosp, eidx)
    p1 = v1[0] + v1[1]
    xb2 = _valign_s2(featT2, p1[:, 0], p1[:, 1])
    v2 = _stage(xb2, 's2', params, zerosp, eidx)
    p2 = v2[0] + v2[1]
    xb3 = _valign_s3(featT3, p2[:, 0], p2[:, 1])
    v3 = _stage(xb3, 's3', params, zerosp, eidx)

    return (v3[0] + v3[1])[:N_VERTS, :3]
